# baseline (speedup 1.0000x reference)
import functools

import jax
import jax.numpy as jnp
from jax import lax
from jax.experimental import pallas as pl
from jax.experimental.pallas import tpu as pltpu
from jax.experimental.pallas import tpu_sc as plsc

SEQ = 10
LOC = 20
ED = 128
HALF = 64
BATCH = 16384

ROWS = BATCH * SEQ
NC, NS = 2, 16
NW = NC * NS
RPW = ROWS // NW
CHUNK = 128
NCH = RPW // CHUNK


def _table_body(tok_ref, w_ref, b_ref, pos_ref, o_ref):
    t = jnp.dot(tok_ref[:], w_ref[:], preferred_element_type=jnp.float32)
    t = t + b_ref[:]
    zeros = jnp.zeros((LOC, HALF), jnp.float32)
    tl = jnp.concatenate([t, zeros], axis=1)
    tr = jnp.concatenate([zeros, t], axis=1)
    g = (tl[None, :, None, :] + tr[None, None, :, :]
         + pos_ref[:][:, None, None, :])
    o_ref[:] = g


def _build_table(tok_table, W, b, pos_table):
    return pl.pallas_call(
        _table_body,
        out_shape=jax.ShapeDtypeStruct((SEQ, LOC, LOC, ED), jnp.float32),
    )(tok_table, W, b.reshape(1, HALF), pos_table)


def _gather_body(g_hbm, q_hbm, out_hbm, qbuf, cidx, buf0, buf1, buf2, buf3,
                 gsem0, gsem1, gsem2, gsem3, ssem0, ssem1, ssem2, ssem3):
    wid = lax.axis_index("s") * NC + lax.axis_index("c")
    rowbase = wid * RPW

    pltpu.sync_copy(q_hbm.at[pl.ds(rowbase * 2, RPW * 2)], qbuf)

    def idx_body(g, carry):
        base = (g // 8) * 256 + (g % 8) * 16
        i0 = qbuf[pl.ds(base, 16)]
        i1 = qbuf[pl.ds(base + 128, 16)]
        s = (rowbase + g * 16) // BATCH
        c = s * (LOC * LOC) + i0 * LOC + i1
        cidx[g // 8, pl.ds((g % 8) * 16, 16)] = c
        return carry

    lax.fori_loop(0, RPW // 16, idx_body, 0)

    def start_gather(t, buf, sem):
        pltpu.make_async_copy(g_hbm.at[cidx.at[t]], buf, sem).start()

    def wait_gather(buf, sem):
        pltpu.make_async_copy(g_hbm.at[cidx.at[0]], buf, sem).wait()

    def start_scatter(t, buf, sem):
        pltpu.make_async_copy(
            buf, out_hbm.at[pl.ds(rowbase + t * CHUNK, CHUNK)], sem).start()

    def wait_scatter(buf, sem):
        pltpu.make_async_copy(
            buf, out_hbm.at[pl.ds(rowbase, CHUNK)], sem).wait()

    bufs = [(buf0, gsem0, ssem0), (buf1, gsem1, ssem1),
            (buf2, gsem2, ssem2), (buf3, gsem3, ssem3)]

    start_gather(0, buf0, gsem0)
    start_gather(1, buf1, gsem1)

    def pipe_body(t4, carry):
        for j in range(4):
            t = t4 * 4 + j
            buf, gsem, ssem = bufs[j]
            nbuf, ngsem, nssem = bufs[(j + 2) % 4]
            wait_gather(buf, gsem)
            start_scatter(t, buf, ssem)

            @pl.when(t < NCH - 2)
            def _():
                @pl.when(t >= 2)
                def _():
                    wait_scatter(nbuf, nssem)

                start_gather(t + 2, nbuf, ngsem)

        return carry

    lax.fori_loop(0, NCH // 4, pipe_body, 0)
    for j in range(4):
        buf, _, ssem = bufs[j]
        wait_scatter(buf, ssem)


def _gather_rows(g_flat, q_flat):
    mesh = plsc.VectorSubcoreMesh(core_axis_name="c", subcore_axis_name="s")
    f = functools.partial(
        pl.kernel,
        mesh=mesh,
        out_type=jax.ShapeDtypeStruct((ROWS, ED), jnp.float32),
        scratch_types=[
            pltpu.VMEM((2 * RPW,), jnp.int32),
            pltpu.VMEM((NCH, CHUNK), jnp.int32),
            pltpu.VMEM((CHUNK, ED), jnp.float32),
            pltpu.VMEM((CHUNK, ED), jnp.float32),
            pltpu.VMEM((CHUNK, ED), jnp.float32),
            pltpu.VMEM((CHUNK, ED), jnp.float32),
        ] + [pltpu.SemaphoreType.DMA] * 8,
    )(_gather_body)
    return f(g_flat, q_flat)


def kernel(inputs, tok_table, W, b, pos_table):
    g = _build_table(tok_table, W, b, pos_table).reshape(SEQ * LOC * LOC, ED)
    q = jnp.transpose(
        inputs.astype(jnp.int32).reshape(BATCH // 128, 128, SEQ, 2),
        (2, 0, 3, 1),
    ).reshape(-1)
    out = _gather_rows(g, q)
    return jnp.transpose(out.reshape(SEQ, BATCH, ED), (1, 0, 2))


# speedup vs baseline: 1.0598x; 1.0598x over previous
import functools

import jax
import jax.numpy as jnp
from jax import lax
from jax.experimental import pallas as pl
from jax.experimental.pallas import tpu as pltpu
from jax.experimental.pallas import tpu_sc as plsc

SEQ = 10
LOC = 20
ED = 128
HALF = 64
BATCH = 16384

ROWS = BATCH * SEQ
NC, NS = 2, 16
NW = NC * NS
RPW = ROWS // NW
CHUNK = 128
NCH = RPW // CHUNK


def _table_body(tok_ref, wt_ref, b_ref, pos_ref, o_ref):
    t = lax.dot_general(tok_ref[:], wt_ref[:], (((1,), (1,)), ((), ())),
                        preferred_element_type=jnp.float32)
    zeros = jnp.zeros((LOC, HALF), jnp.float32)
    tl = jnp.concatenate([t, zeros], axis=1)
    tr = jnp.concatenate([zeros, t], axis=1)
    p_iota = lax.broadcasted_iota(jnp.int32, (LOC * LOC, LOC), 0)
    k_iota = lax.broadcasted_iota(jnp.int32, (LOC * LOC, LOC), 1)
    onehot = (p_iota // LOC == k_iota).astype(jnp.float32)
    s400 = lax.dot_general(onehot, tl, (((1,), (0,)), ((), ())),
                           preferred_element_type=jnp.float32)
    s400 = s400 + jnp.concatenate([tr] * LOC, axis=0)
    posb = pos_ref[:] + jnp.concatenate([b_ref[:], b_ref[:]], axis=1)
    o_ref[:] = s400[None, :, :] + posb[:, None, :]


def _build_table(tok_table, W, b, pos_table):
    return pl.pallas_call(
        _table_body,
        out_shape=jax.ShapeDtypeStruct((SEQ, LOC * LOC, ED), jnp.float32),
    )(tok_table, jnp.transpose(W), b.reshape(1, HALF), pos_table)


def _gather_body(g_hbm, q_hbm, out_hbm, qbuf, cidx, buf0, buf1, buf2, buf3,
                 gsem0, gsem1, gsem2, gsem3, ssem0, ssem1, ssem2, ssem3):
    wid = lax.axis_index("s") * NC + lax.axis_index("c")
    rowbase = wid * RPW

    pltpu.sync_copy(q_hbm.at[pl.ds(rowbase * 2, RPW * 2)], qbuf)

    def idx_body(g, carry):
        base = (g // 8) * 256 + (g % 8) * 16
        i0 = qbuf[pl.ds(base, 16)]
        i1 = qbuf[pl.ds(base + 128, 16)]
        s = (rowbase + g * 16) // BATCH
        c = s * (LOC * LOC) + i0 * LOC + i1
        cidx[g // 8, pl.ds((g % 8) * 16, 16)] = c
        return carry

    lax.fori_loop(0, RPW // 16, idx_body, 0)

    def start_gather(t, buf, sem):
        pltpu.make_async_copy(g_hbm.at[cidx.at[t]], buf, sem).start()

    def wait_gather(buf, sem):
        pltpu.make_async_copy(g_hbm.at[cidx.at[0]], buf, sem).wait()

    def start_scatter(t, buf, sem):
        pltpu.make_async_copy(
            buf, out_hbm.at[pl.ds(rowbase + t * CHUNK, CHUNK)], sem).start()

    def wait_scatter(buf, sem):
        pltpu.make_async_copy(
            buf, out_hbm.at[pl.ds(rowbase, CHUNK)], sem).wait()

    bufs = [(buf0, gsem0, ssem0), (buf1, gsem1, ssem1),
            (buf2, gsem2, ssem2), (buf3, gsem3, ssem3)]

    start_gather(0, buf0, gsem0)
    start_gather(1, buf1, gsem1)

    def pipe_body(t4, carry):
        for j in range(4):
            t = t4 * 4 + j
            buf, gsem, ssem = bufs[j]
            nbuf, ngsem, nssem = bufs[(j + 2) % 4]
            wait_gather(buf, gsem)
            start_scatter(t, buf, ssem)

            @pl.when(t < NCH - 2)
            def _():
                @pl.when(t >= 2)
                def _():
                    wait_scatter(nbuf, nssem)

                start_gather(t + 2, nbuf, ngsem)

        return carry

    lax.fori_loop(0, NCH // 4, pipe_body, 0)
    for j in range(4):
        buf, _, ssem = bufs[j]
        wait_scatter(buf, ssem)


def _gather_rows(g_flat, q_flat):
    mesh = plsc.VectorSubcoreMesh(core_axis_name="c", subcore_axis_name="s")
    f = functools.partial(
        pl.kernel,
        mesh=mesh,
        out_type=jax.ShapeDtypeStruct((ROWS, ED), jnp.float32),
        scratch_types=[
            pltpu.VMEM((2 * RPW,), jnp.int32),
            pltpu.VMEM((NCH, CHUNK), jnp.int32),
            pltpu.VMEM((CHUNK, ED), jnp.float32),
            pltpu.VMEM((CHUNK, ED), jnp.float32),
            pltpu.VMEM((CHUNK, ED), jnp.float32),
            pltpu.VMEM((CHUNK, ED), jnp.float32),
        ] + [pltpu.SemaphoreType.DMA] * 8,
    )(_gather_body)
    return f(g_flat, q_flat)


def kernel(inputs, tok_table, W, b, pos_table):
    g = _build_table(tok_table, W, b, pos_table).reshape(SEQ * LOC * LOC, ED)
    q = jnp.transpose(
        inputs.astype(jnp.int32).reshape(BATCH // 128, 128, SEQ, 2),
        (2, 0, 3, 1),
    ).reshape(-1)
    out = _gather_rows(g, q)
    return jnp.transpose(out.reshape(SEQ, BATCH, ED), (1, 0, 2))
